# baseline (speedup 1.0000x reference)
import functools

import jax
import jax.numpy as jnp
from jax import lax
from jax.experimental import pallas as pl
from jax.experimental.pallas import tpu as pltpu
from jax.experimental.pallas import tpu_sc as plsc


def _embed_kernel(F, V, D, B):
    NC, NS, L = 2, 16, 16
    NW = NC * NS
    R = 128
    ROWS_W = B * F // NW
    N_CHUNK = ROWS_W // R
    PACK = 128 // D
    assert ROWS_W % R == 0 and N_CHUNK % 2 == 0

    mesh = plsc.VectorSubcoreMesh(core_axis_name="c", subcore_axis_name="s")

    @functools.partial(
        pl.kernel,
        out_type=jax.ShapeDtypeStruct((B * F, D), jnp.float32),
        mesh=mesh,
        scratch_types=[
            pltpu.VMEM((N_CHUNK, R), jnp.int32),
            pltpu.VMEM((R,), jnp.int32),
            pltpu.VMEM((R,), jnp.int32),
            pltpu.VMEM((R, 128), jnp.float32),
            pltpu.VMEM((R, 128), jnp.float32),
            pltpu.VMEM((R, D), jnp.float32),
            pltpu.VMEM((R, D), jnp.float32),
            pltpu.SemaphoreType.DMA,
            pltpu.SemaphoreType.DMA,
            pltpu.SemaphoreType.DMA,
            pltpu.SemaphoreType.DMA,
        ],
        compiler_params=pltpu.CompilerParams(use_tc_tiling_on_sc=True,
                                             needs_layout_passes=False),
    )
    def k(idx_hbm, tab_hbm, out_hbm, idx_v, ri0, ri1, fb0, fb1, im0, im1,
          sg0, sg1, sw0, sw1):
        wid = lax.axis_index("s") * NC + lax.axis_index("c")
        row0 = wid * ROWS_W
        ri = (ri0, ri1)
        fb = (fb0, fb1)
        im = (im0, im1)
        sg = (sg0, sg1)
        sw = (sw0, sw1)

        pltpu.sync_copy(idx_hbm.at[pl.ds(wid * N_CHUNK, N_CHUNK)], idx_v)

        iota = lax.iota(jnp.int32, L)

        def transform(c, r):
            def tj(j, carry):
                pos = j * L
                x = plsc.load_gather(idx_v, [jnp.full((L,), c, jnp.int32),
                                             pos + iota])
                plsc.store_scatter(ri[r], [pos + iota],
                                   lax.shift_right_logical(x, 2))
                return carry
            lax.fori_loop(0, R // L, tj, 0)

        def fire_g(c, r):
            pltpu.async_copy(tab_hbm.at[ri[r]], fb[r], sg[r])

        def wait_g(r):
            pltpu.make_async_copy(tab_hbm.at[pl.ds(0, R)], fb[r], sg[r]).wait()

        def select(c, r):
            def sj(j, carry):
                pos = j * L
                x = plsc.load_gather(idx_v, [jnp.full((L,), c, jnp.int32),
                                             pos + iota])
                lane0 = lax.mul(lax.bitwise_and(x, PACK - 1), D)
                rvec = pos + iota
                for dd in range(D):
                    val = plsc.load_gather(fb[r], [rvec, lane0 + dd])
                    plsc.store_scatter(im[r],
                                       [rvec, jnp.full((L,), dd, jnp.int32)],
                                       val)
                return carry
            lax.fori_loop(0, R // L, sj, 0)

        def fire_w(c, r):
            pltpu.async_copy(im[r], out_hbm.at[pl.ds(row0 + c * R, R)], sw[r])

        def wait_w(r):
            pltpu.make_async_copy(im[r], out_hbm.at[pl.ds(0, R)], sw[r]).wait()

        transform(0, 0)
        fire_g(0, 0)
        transform(1, 1)
        fire_g(1, 1)

        def body(i, carry):
            for r in range(2):
                c = 2 * i + r
                wait_g(r)

                @pl.when(i >= 1)
                def _():
                    wait_w(r)
                select(c, r)
                fire_w(c, r)

                @pl.when(i < N_CHUNK // 2 - 1)
                def _():
                    transform(c + 2, r)
                    fire_g(c + 2, r)
            return carry

        lax.fori_loop(0, N_CHUNK // 2, body, 0)
        wait_w(0)
        wait_w(1)

    return k


def kernel(indices, tables):
    F, B = indices.shape
    _, V, D = tables.shape
    tab128 = tables.reshape(F * V * D // 128, 128)
    offs = (jnp.arange(F, dtype=jnp.int32) * V)[:, None]
    idx2d = (indices.astype(jnp.int32) + offs).T.reshape(-1, 128)
    out = _embed_kernel(F, V, D, B)(idx2d, tab128)
    return out.reshape(B, F * D)


# speedup vs baseline: 1.4128x; 1.4128x over previous
import functools

import jax
import jax.numpy as jnp
from jax import lax
from jax.experimental import pallas as pl
from jax.experimental.pallas import tpu as pltpu
from jax.experimental.pallas import tpu_sc as plsc


def _embed_kernel(F, V, D, B):
    NC, NS = 2, 16
    NW = NC * NS
    B_PER_W = B // NW
    B_CHUNK = 64
    N_CHUNK = B_PER_W // B_CHUNK
    assert B % (NW * B_CHUNK) == 0 and N_CHUNK % 2 == 0

    mesh = plsc.VectorSubcoreMesh(core_axis_name="c", subcore_axis_name="s")

    @functools.partial(
        pl.kernel,
        out_type=jax.ShapeDtypeStruct((B, F * D), jnp.float32),
        mesh=mesh,
        scratch_types=[
            pltpu.VMEM((F, B_PER_W), jnp.int32),
            pltpu.VMEM((F, B_CHUNK, D), jnp.float32),
            pltpu.VMEM((F, B_CHUNK, D), jnp.float32),
            pltpu.SemaphoreType.DMA,
            pltpu.SemaphoreType.DMA,
            pltpu.SemaphoreType.DMA,
            pltpu.SemaphoreType.DMA,
        ],
        compiler_params=pltpu.CompilerParams(use_tc_tiling_on_sc=False),
    )
    def k(idx_hbm, tab_hbm, out_hbm, idx_v, rows0, rows1, sg0, sg1, sw0, sw1):
        wid = lax.axis_index("s") * NC + lax.axis_index("c")
        b0 = wid * B_PER_W
        rows = (rows0, rows1)
        sg = (sg0, sg1)
        sw = (sw0, sw1)

        pltpu.sync_copy(idx_hbm.at[:, pl.ds(b0, B_PER_W)], idx_v)

        def fire_g(c, r):
            for f in range(F):
                pltpu.async_copy(
                    tab_hbm.at[f].at[idx_v.at[f, pl.ds(c * B_CHUNK, B_CHUNK)]],
                    rows[r].at[f],
                    sg[r])

        def wait_g(r):
            pltpu.make_async_copy(
                tab_hbm.at[:, pl.ds(0, B_CHUNK), :], rows[r], sg[r]).wait()

        def fire_w(c, r):
            for f in range(F):
                pltpu.async_copy(
                    rows[r].at[f],
                    out_hbm.at[pl.ds(b0 + c * B_CHUNK, B_CHUNK),
                               pl.ds(f * D, D)],
                    sw[r])

        def wait_w(r):
            pltpu.make_async_copy(
                tab_hbm.at[:, pl.ds(0, B_CHUNK), :], rows[r], sw[r]).wait()

        fire_g(0, 0)

        def body(i, carry):
            c = 2 * i
            wait_g(0)
            fire_w(c, 0)

            @pl.when(i >= 1)
            def _():
                wait_w(1)
            fire_g(c + 1, 1)

            wait_g(1)
            fire_w(c + 1, 1)

            @pl.when(i < N_CHUNK // 2 - 1)
            def _():
                wait_w(0)
                fire_g(c + 2, 0)
            return carry

        lax.fori_loop(0, N_CHUNK // 2, body, 0)
        wait_w(0)
        wait_w(1)

    return k


def kernel(indices, tables):
    F, B = indices.shape
    _, V, D = tables.shape
    return _embed_kernel(F, V, D, B)(indices.astype(jnp.int32), tables)
